# Initial kernel scaffold; baseline (speedup 1.0000x reference)
import jax
import jax.numpy as jnp
from jax.experimental import pallas as pl


def kernel(x, gcn_mat, sct_mat, W0, W1, W2, W3, W4, W5, b0, b1, b2, b3, b4, b5):
    raise NotImplementedError("write your pallas kernel here")


# speedup vs baseline: 1.3956x; 1.3956x over previous
import functools

import jax
import jax.numpy as jnp
from jax.experimental import pallas as pl
from jax.experimental.pallas import tpu as pltpu

_BM = 512
_BK = 2048


def _mm_kernel(a_ref, x_ref, o_ref, *, nk, krem):
    k = pl.program_id(1)

    @pl.when(k == 0)
    def _():
        o_ref[...] = jnp.zeros_like(o_ref)

    @pl.when(k < nk - 1)
    def _():
        o_ref[...] += jnp.dot(a_ref[...], x_ref[...],
                              preferred_element_type=jnp.float32)

    @pl.when(k == nk - 1)
    def _():
        a = a_ref[...]
        ak = jax.lax.broadcasted_iota(jnp.int32, a.shape, 1)
        a = jnp.where(ak < krem, a, 0.0)
        xv = x_ref[...]
        xk = jax.lax.broadcasted_iota(jnp.int32, xv.shape, 0)
        xv = jnp.where(xk < krem, xv, 0.0)
        o_ref[...] += jnp.dot(a, xv, preferred_element_type=jnp.float32)


def _mm(a, xmat):
    n = a.shape[0]
    w = xmat.shape[1]
    nm = pl.cdiv(n, _BM)
    nk = pl.cdiv(n, _BK)
    krem = n - (nk - 1) * _BK
    return pl.pallas_call(
        functools.partial(_mm_kernel, nk=nk, krem=krem),
        grid=(nm, nk),
        in_specs=[
            pl.BlockSpec((_BM, _BK), lambda i, k: (i, k)),
            pl.BlockSpec((_BK, w), lambda i, k: (k, 0)),
        ],
        out_specs=pl.BlockSpec((_BM, w), lambda i, k: (i, 0)),
        out_shape=jax.ShapeDtypeStruct((n, w), jnp.float32),
        compiler_params=pltpu.CompilerParams(
            dimension_semantics=("parallel", "arbitrary"),
        ),
    )(a, xmat)


def _proj_kernel(x_ref, w_ref, o_ref):
    o_ref[...] = jax.lax.dot_general(
        x_ref[...], w_ref[...],
        dimension_numbers=(((1,), (1,)), ((), ())),
        preferred_element_type=jnp.float32)


def _proj(x, wcat):
    n, d = x.shape
    h = wcat.shape[0]
    nm = pl.cdiv(n, _BM)
    return pl.pallas_call(
        _proj_kernel,
        grid=(nm,),
        in_specs=[
            pl.BlockSpec((_BM, d), lambda i: (i, 0)),
            pl.BlockSpec((h, d), lambda i: (0, 0)),
        ],
        out_specs=pl.BlockSpec((_BM, h), lambda i: (i, 0)),
        out_shape=jax.ShapeDtypeStruct((n, h), jnp.float32),
    )(x, wcat)


def _combine_kernel(g1_ref, g2_ref, g3_ref, s1_ref, s2_ref, s4_ref, s8_ref,
                    b_ref, o_ref):
    b = b_ref[...]
    o_ref[:, 0:16] = jnp.maximum(g1_ref[:, 0:16] + b[:, 0:16], 0.0)
    o_ref[:, 16:32] = jnp.maximum(g2_ref[:, 0:16] + b[:, 16:32], 0.0)
    o_ref[:, 32:48] = jnp.maximum(g3_ref[...] + b[:, 32:48], 0.0)
    o_ref[:, 48:64] = jnp.maximum(
        s1_ref[:, 0:16] - s2_ref[:, 0:16] + b[:, 48:64], 0.0)
    o_ref[:, 64:80] = jnp.maximum(
        s2_ref[:, 16:32] - s4_ref[:, 0:16] + b[:, 64:80], 0.0)
    o_ref[:, 80:96] = jnp.maximum(
        s4_ref[:, 16:32] - s8_ref[...] + b[:, 80:96], 0.0)


def _combine(g1, g2, g3, s1, s2, s4, s8, bcat):
    n = g1.shape[0]
    nm = pl.cdiv(n, _BM)
    args = (g1, g2, g3, s1, s2, s4, s8)
    in_specs = [pl.BlockSpec((_BM, a.shape[1]), lambda i: (i, 0))
                for a in args]
    in_specs.append(pl.BlockSpec((1, 96), lambda i: (0, 0)))
    return pl.pallas_call(
        _combine_kernel,
        grid=(nm,),
        in_specs=in_specs,
        out_specs=pl.BlockSpec((_BM, 96), lambda i: (i, 0)),
        out_shape=jax.ShapeDtypeStruct((n, 96), jnp.float32),
    )(*args, bcat)


def kernel(x, gcn_mat, sct_mat, W0, W1, W2, W3, W4, W5,
           b0, b1, b2, b3, b4, b5):
    wcat = jnp.concatenate([W0, W1, W2, W3, W4, W5], axis=0)
    bcat = jnp.concatenate([b0, b1, b2, b3, b4, b5], axis=1)

    h = _proj(x, wcat)

    g1 = _mm(gcn_mat, h[:, 0:48])
    g2 = _mm(gcn_mat, g1[:, 16:48])
    g3 = _mm(gcn_mat, g2[:, 16:32])

    s1 = _mm(sct_mat, h[:, 48:96])
    s2 = _mm(sct_mat, s1)
    s3 = _mm(sct_mat, s2[:, 16:48])
    s4 = _mm(sct_mat, s3)
    s5 = _mm(sct_mat, s4[:, 16:32])
    s6 = _mm(sct_mat, s5)
    s7 = _mm(sct_mat, s6)
    s8 = _mm(sct_mat, s7)

    return _combine(g1, g2, g3, s1, s2, s4, s8, bcat)


# speedup vs baseline: 1.5146x; 1.0853x over previous
import functools

import jax
import jax.numpy as jnp
from jax.experimental import pallas as pl
from jax.experimental.pallas import tpu as pltpu

_BM = 512
_BK = 2048


def _mm_cast_kernel(a_ref, x_ref, o_ref, abf_ref, *, nk, krem):
    k = pl.program_id(1)

    @pl.when(k == 0)
    def _():
        o_ref[...] = jnp.zeros_like(o_ref)

    @pl.when(k < nk - 1)
    def _():
        abf = a_ref[...].astype(jnp.bfloat16)
        abf_ref[...] = abf
        o_ref[...] += jnp.dot(abf, x_ref[...].astype(jnp.bfloat16),
                              preferred_element_type=jnp.float32)

    @pl.when(k == nk - 1)
    def _():
        a = a_ref[...]
        ak = jax.lax.broadcasted_iota(jnp.int32, a.shape, 1)
        a = jnp.where(ak < krem, a, 0.0)
        abf = a.astype(jnp.bfloat16)
        abf_ref[...] = abf
        xv = x_ref[...]
        xk = jax.lax.broadcasted_iota(jnp.int32, xv.shape, 0)
        xv = jnp.where(xk < krem, xv, 0.0)
        o_ref[...] += jnp.dot(abf, xv.astype(jnp.bfloat16),
                              preferred_element_type=jnp.float32)


def _mm_cast(a, xmat):
    n = a.shape[0]
    w = xmat.shape[1]
    nm = pl.cdiv(n, _BM)
    nk = pl.cdiv(n, _BK)
    krem = n - (nk - 1) * _BK
    return pl.pallas_call(
        functools.partial(_mm_cast_kernel, nk=nk, krem=krem),
        grid=(nm, nk),
        in_specs=[
            pl.BlockSpec((_BM, _BK), lambda i, k: (i, k)),
            pl.BlockSpec((_BK, w), lambda i, k: (k, 0)),
        ],
        out_specs=[
            pl.BlockSpec((_BM, w), lambda i, k: (i, 0)),
            pl.BlockSpec((_BM, _BK), lambda i, k: (i, k)),
        ],
        out_shape=[
            jax.ShapeDtypeStruct((n, w), jnp.float32),
            jax.ShapeDtypeStruct((n, n), jnp.bfloat16),
        ],
        compiler_params=pltpu.CompilerParams(
            dimension_semantics=("parallel", "arbitrary"),
        ),
    )(a, xmat)


def _mm_bf_kernel(a_ref, x_ref, o_ref, *, nk, krem):
    k = pl.program_id(1)

    @pl.when(k == 0)
    def _():
        o_ref[...] = jnp.zeros_like(o_ref)

    @pl.when(k < nk - 1)
    def _():
        o_ref[...] += jnp.dot(a_ref[...], x_ref[...].astype(jnp.bfloat16),
                              preferred_element_type=jnp.float32)

    @pl.when(k == nk - 1)
    def _():
        xv = x_ref[...]
        xk = jax.lax.broadcasted_iota(jnp.int32, xv.shape, 0)
        xv = jnp.where(xk < krem, xv, 0.0)
        a = a_ref[...]
        ak = jax.lax.broadcasted_iota(jnp.int32, a.shape, 1)
        a = jnp.where(ak < krem, a, jnp.bfloat16(0))
        o_ref[...] += jnp.dot(a, xv.astype(jnp.bfloat16),
                              preferred_element_type=jnp.float32)


def _mm_bf(a, xmat):
    n = a.shape[0]
    w = xmat.shape[1]
    nm = pl.cdiv(n, _BM)
    nk = pl.cdiv(n, _BK)
    krem = n - (nk - 1) * _BK
    return pl.pallas_call(
        functools.partial(_mm_bf_kernel, nk=nk, krem=krem),
        grid=(nm, nk),
        in_specs=[
            pl.BlockSpec((_BM, _BK), lambda i, k: (i, k)),
            pl.BlockSpec((_BK, w), lambda i, k: (k, 0)),
        ],
        out_specs=pl.BlockSpec((_BM, w), lambda i, k: (i, 0)),
        out_shape=jax.ShapeDtypeStruct((n, w), jnp.float32),
        compiler_params=pltpu.CompilerParams(
            dimension_semantics=("parallel", "arbitrary"),
        ),
    )(a, xmat)


def _proj_kernel(x_ref, w_ref, o_ref):
    o_ref[...] = jax.lax.dot_general(
        x_ref[...], w_ref[...],
        dimension_numbers=(((1,), (1,)), ((), ())),
        preferred_element_type=jnp.float32)


def _proj(x, wcat):
    n, d = x.shape
    h = wcat.shape[0]
    nm = pl.cdiv(n, _BM)
    return pl.pallas_call(
        _proj_kernel,
        grid=(nm,),
        in_specs=[
            pl.BlockSpec((_BM, d), lambda i: (i, 0)),
            pl.BlockSpec((h, d), lambda i: (0, 0)),
        ],
        out_specs=pl.BlockSpec((_BM, h), lambda i: (i, 0)),
        out_shape=jax.ShapeDtypeStruct((n, h), jnp.float32),
    )(x, wcat)


def _combine_kernel(g1_ref, g2_ref, g3_ref, s1_ref, s2_ref, s4_ref, s8_ref,
                    b_ref, o_ref):
    b = b_ref[...]
    o_ref[:, 0:16] = jnp.maximum(g1_ref[:, 0:16] + b[:, 0:16], 0.0)
    o_ref[:, 16:32] = jnp.maximum(g2_ref[:, 0:16] + b[:, 16:32], 0.0)
    o_ref[:, 32:48] = jnp.maximum(g3_ref[...] + b[:, 32:48], 0.0)
    o_ref[:, 48:64] = jnp.maximum(
        s1_ref[:, 0:16] - s2_ref[:, 0:16] + b[:, 48:64], 0.0)
    o_ref[:, 64:80] = jnp.maximum(
        s2_ref[:, 16:32] - s4_ref[:, 0:16] + b[:, 64:80], 0.0)
    o_ref[:, 80:96] = jnp.maximum(
        s4_ref[:, 16:32] - s8_ref[...] + b[:, 80:96], 0.0)


def _combine(g1, g2, g3, s1, s2, s4, s8, bcat):
    n = g1.shape[0]
    nm = pl.cdiv(n, _BM)
    args = (g1, g2, g3, s1, s2, s4, s8)
    in_specs = [pl.BlockSpec((_BM, a.shape[1]), lambda i: (i, 0))
                for a in args]
    in_specs.append(pl.BlockSpec((1, 96), lambda i: (0, 0)))
    return pl.pallas_call(
        _combine_kernel,
        grid=(nm,),
        in_specs=in_specs,
        out_specs=pl.BlockSpec((_BM, 96), lambda i: (i, 0)),
        out_shape=jax.ShapeDtypeStruct((n, 96), jnp.float32),
    )(*args, bcat)


def kernel(x, gcn_mat, sct_mat, W0, W1, W2, W3, W4, W5,
           b0, b1, b2, b3, b4, b5):
    wcat = jnp.concatenate([W0, W1, W2, W3, W4, W5], axis=0)
    bcat = jnp.concatenate([b0, b1, b2, b3, b4, b5], axis=1)

    h = _proj(x, wcat)

    g1, gcn_bf = _mm_cast(gcn_mat, h[:, 0:48])
    g2 = _mm_bf(gcn_bf, g1[:, 16:48])
    g3 = _mm_bf(gcn_bf, g2[:, 16:32])

    s1, sct_bf = _mm_cast(sct_mat, h[:, 48:96])
    s2 = _mm_bf(sct_bf, s1)
    s3 = _mm_bf(sct_bf, s2[:, 16:48])
    s4 = _mm_bf(sct_bf, s3)
    s5 = _mm_bf(sct_bf, s4[:, 16:32])
    s6 = _mm_bf(sct_bf, s5)
    s7 = _mm_bf(sct_bf, s6)
    s8 = _mm_bf(sct_bf, s7)

    return _combine(g1, g2, g3, s1, s2, s4, s8, bcat)


# speedup vs baseline: 2.3886x; 1.5770x over previous
import jax
import jax.numpy as jnp
from jax.experimental import pallas as pl
from jax.experimental.pallas import tpu as pltpu

_BM_CAST = 256
_BM = 512


def _mm_cast_kernel(a_ref, x_ref, o_ref, abf_ref):
    abf = a_ref[...].astype(jnp.bfloat16)
    abf_ref[...] = abf
    o_ref[...] = jnp.dot(abf, x_ref[...].astype(jnp.bfloat16),
                         preferred_element_type=jnp.float32)


def _mm_cast(a, xmat):
    n = a.shape[0]
    w = xmat.shape[1]
    return pl.pallas_call(
        _mm_cast_kernel,
        grid=(pl.cdiv(n, _BM_CAST),),
        in_specs=[
            pl.BlockSpec((_BM_CAST, n), lambda i: (i, 0)),
            pl.BlockSpec((n, w), lambda i: (0, 0)),
        ],
        out_specs=[
            pl.BlockSpec((_BM_CAST, w), lambda i: (i, 0)),
            pl.BlockSpec((_BM_CAST, n), lambda i: (i, 0)),
        ],
        out_shape=[
            jax.ShapeDtypeStruct((n, w), jnp.float32),
            jax.ShapeDtypeStruct((n, n), jnp.bfloat16),
        ],
        compiler_params=pltpu.CompilerParams(
            dimension_semantics=("arbitrary",),
        ),
    )(a, xmat)


def _mm_bf_kernel(a_ref, x_ref, o_ref):
    o_ref[...] = jnp.dot(a_ref[...], x_ref[...].astype(jnp.bfloat16),
                         preferred_element_type=jnp.float32)


def _mm_bf(a, xmat):
    n = a.shape[0]
    w = xmat.shape[1]
    return pl.pallas_call(
        _mm_bf_kernel,
        grid=(pl.cdiv(n, _BM),),
        in_specs=[
            pl.BlockSpec((_BM, n), lambda i: (i, 0)),
            pl.BlockSpec((n, w), lambda i: (0, 0)),
        ],
        out_specs=pl.BlockSpec((_BM, w), lambda i: (i, 0)),
        out_shape=jax.ShapeDtypeStruct((n, w), jnp.float32),
        compiler_params=pltpu.CompilerParams(
            dimension_semantics=("arbitrary",),
        ),
    )(a, xmat)


def _proj_kernel(x_ref, w_ref, o_ref):
    o_ref[...] = jax.lax.dot_general(
        x_ref[...], w_ref[...],
        dimension_numbers=(((1,), (1,)), ((), ())),
        preferred_element_type=jnp.float32)


def _proj(x, wcat):
    n, d = x.shape
    h = wcat.shape[0]
    return pl.pallas_call(
        _proj_kernel,
        grid=(pl.cdiv(n, _BM),),
        in_specs=[
            pl.BlockSpec((_BM, d), lambda i: (i, 0)),
            pl.BlockSpec((h, d), lambda i: (0, 0)),
        ],
        out_specs=pl.BlockSpec((_BM, h), lambda i: (i, 0)),
        out_shape=jax.ShapeDtypeStruct((n, h), jnp.float32),
    )(x, wcat)


def _combine_kernel(g1_ref, g2_ref, g3_ref, s1_ref, s2_ref, s4_ref, s8_ref,
                    b_ref, o_ref):
    b = b_ref[...]
    o_ref[:, 0:16] = jnp.maximum(g1_ref[:, 0:16] + b[:, 0:16], 0.0)
    o_ref[:, 16:32] = jnp.maximum(g2_ref[:, 0:16] + b[:, 16:32], 0.0)
    o_ref[:, 32:48] = jnp.maximum(g3_ref[...] + b[:, 32:48], 0.0)
    o_ref[:, 48:64] = jnp.maximum(
        s1_ref[:, 0:16] - s2_ref[:, 0:16] + b[:, 48:64], 0.0)
    o_ref[:, 64:80] = jnp.maximum(
        s2_ref[:, 16:32] - s4_ref[:, 0:16] + b[:, 64:80], 0.0)
    o_ref[:, 80:96] = jnp.maximum(
        s4_ref[:, 16:32] - s8_ref[...] + b[:, 80:96], 0.0)


def _combine(g1, g2, g3, s1, s2, s4, s8, bcat):
    n = g1.shape[0]
    args = (g1, g2, g3, s1, s2, s4, s8)
    in_specs = [pl.BlockSpec((_BM, a.shape[1]), lambda i: (i, 0))
                for a in args]
    in_specs.append(pl.BlockSpec((1, 96), lambda i: (0, 0)))
    return pl.pallas_call(
        _combine_kernel,
        grid=(pl.cdiv(n, _BM),),
        in_specs=in_specs,
        out_specs=pl.BlockSpec((_BM, 96), lambda i: (i, 0)),
        out_shape=jax.ShapeDtypeStruct((n, 96), jnp.float32),
    )(*args, bcat)


def kernel(x, gcn_mat, sct_mat, W0, W1, W2, W3, W4, W5,
           b0, b1, b2, b3, b4, b5):
    wcat = jnp.concatenate([W0, W1, W2, W3, W4, W5], axis=0)
    bcat = jnp.concatenate([b0, b1, b2, b3, b4, b5], axis=1)

    h = _proj(x, wcat)

    g1, gcn_bf = _mm_cast(gcn_mat, h[:, 0:48])
    g2 = _mm_bf(gcn_bf, g1[:, 16:48])
    g3 = _mm_bf(gcn_bf, g2[:, 16:32])

    s1, sct_bf = _mm_cast(sct_mat, h[:, 48:96])
    s2 = _mm_bf(sct_bf, s1)
    s3 = _mm_bf(sct_bf, s2[:, 16:48])
    s4 = _mm_bf(sct_bf, s3)
    s5 = _mm_bf(sct_bf, s4[:, 16:32])
    s6 = _mm_bf(sct_bf, s5)
    s7 = _mm_bf(sct_bf, s6)
    s8 = _mm_bf(sct_bf, s7)

    return _combine(g1, g2, g3, s1, s2, s4, s8, bcat)


# speedup vs baseline: 2.4862x; 1.0408x over previous
import jax
import jax.numpy as jnp
from jax.experimental import pallas as pl
from jax.experimental.pallas import tpu as pltpu

_BM_CAST = 256
_BM = 1024


def _mm_cast_kernel(a_ref, x_ref, o_ref, abf_ref):
    abf = a_ref[...].astype(jnp.bfloat16)
    abf_ref[...] = abf
    o_ref[...] = jnp.dot(abf, x_ref[...].astype(jnp.bfloat16),
                         preferred_element_type=jnp.float32)


def _mm_cast(a, xmat):
    n = a.shape[0]
    w = xmat.shape[1]
    return pl.pallas_call(
        _mm_cast_kernel,
        grid=(pl.cdiv(n, _BM_CAST),),
        in_specs=[
            pl.BlockSpec((_BM_CAST, n), lambda i: (i, 0)),
            pl.BlockSpec((n, w), lambda i: (0, 0)),
        ],
        out_specs=[
            pl.BlockSpec((_BM_CAST, w), lambda i: (i, 0)),
            pl.BlockSpec((_BM_CAST, n), lambda i: (i, 0)),
        ],
        out_shape=[
            jax.ShapeDtypeStruct((n, w), jnp.float32),
            jax.ShapeDtypeStruct((n, n), jnp.bfloat16),
        ],
        compiler_params=pltpu.CompilerParams(
            dimension_semantics=("arbitrary",),
        ),
    )(a, xmat)


def _mm_bf_kernel(a_ref, x_ref, o_ref):
    o_ref[...] = jnp.dot(a_ref[...], x_ref[...].astype(jnp.bfloat16),
                         preferred_element_type=jnp.float32)


def _mm_bf(a, xmat):
    n = a.shape[0]
    w = xmat.shape[1]
    return pl.pallas_call(
        _mm_bf_kernel,
        grid=(pl.cdiv(n, _BM),),
        in_specs=[
            pl.BlockSpec((_BM, n), lambda i: (i, 0)),
            pl.BlockSpec((n, w), lambda i: (0, 0)),
        ],
        out_specs=pl.BlockSpec((_BM, w), lambda i: (i, 0)),
        out_shape=jax.ShapeDtypeStruct((n, w), jnp.float32),
        compiler_params=pltpu.CompilerParams(
            dimension_semantics=("arbitrary",),
        ),
    )(a, xmat)


def _proj_kernel(x_ref, w_ref, o_ref):
    o_ref[...] = jax.lax.dot_general(
        x_ref[...], w_ref[...],
        dimension_numbers=(((1,), (1,)), ((), ())),
        preferred_element_type=jnp.float32)


def _proj(x, wcat):
    n, d = x.shape
    h = wcat.shape[0]
    return pl.pallas_call(
        _proj_kernel,
        grid=(pl.cdiv(n, _BM),),
        in_specs=[
            pl.BlockSpec((_BM, d), lambda i: (i, 0)),
            pl.BlockSpec((h, d), lambda i: (0, 0)),
        ],
        out_specs=pl.BlockSpec((_BM, h), lambda i: (i, 0)),
        out_shape=jax.ShapeDtypeStruct((n, h), jnp.float32),
    )(x, wcat)


def _combine_kernel(g1_ref, g2_ref, g3_ref, s1_ref, s2_ref, s4_ref, s8_ref,
                    b_ref, o_ref):
    b = b_ref[...]
    o_ref[:, 0:16] = jnp.maximum(g1_ref[:, 0:16] + b[:, 0:16], 0.0)
    o_ref[:, 16:32] = jnp.maximum(g2_ref[:, 0:16] + b[:, 16:32], 0.0)
    o_ref[:, 32:48] = jnp.maximum(g3_ref[...] + b[:, 32:48], 0.0)
    o_ref[:, 48:64] = jnp.maximum(
        s1_ref[:, 0:16] - s2_ref[:, 0:16] + b[:, 48:64], 0.0)
    o_ref[:, 64:80] = jnp.maximum(
        s2_ref[:, 16:32] - s4_ref[:, 0:16] + b[:, 64:80], 0.0)
    o_ref[:, 80:96] = jnp.maximum(
        s4_ref[:, 16:32] - s8_ref[...] + b[:, 80:96], 0.0)


def _combine(g1, g2, g3, s1, s2, s4, s8, bcat):
    n = g1.shape[0]
    args = (g1, g2, g3, s1, s2, s4, s8)
    in_specs = [pl.BlockSpec((_BM, a.shape[1]), lambda i: (i, 0))
                for a in args]
    in_specs.append(pl.BlockSpec((1, 96), lambda i: (0, 0)))
    return pl.pallas_call(
        _combine_kernel,
        grid=(pl.cdiv(n, _BM),),
        in_specs=in_specs,
        out_specs=pl.BlockSpec((_BM, 96), lambda i: (i, 0)),
        out_shape=jax.ShapeDtypeStruct((n, 96), jnp.float32),
    )(*args, bcat)


def kernel(x, gcn_mat, sct_mat, W0, W1, W2, W3, W4, W5,
           b0, b1, b2, b3, b4, b5):
    wcat = jnp.concatenate([W0, W1, W2, W3, W4, W5], axis=0)
    bcat = jnp.concatenate([b0, b1, b2, b3, b4, b5], axis=1)

    h = _proj(x, wcat)

    g1, gcn_bf = _mm_cast(gcn_mat, h[:, 0:48])
    g2 = _mm_bf(gcn_bf, g1[:, 16:48])
    g3 = _mm_bf(gcn_bf, g2[:, 16:32])

    s1, sct_bf = _mm_cast(sct_mat, h[:, 48:96])
    s2 = _mm_bf(sct_bf, s1)
    s3 = _mm_bf(sct_bf, s2[:, 16:48])
    s4 = _mm_bf(sct_bf, s3)
    s5 = _mm_bf(sct_bf, s4[:, 16:32])
    s6 = _mm_bf(sct_bf, s5)
    s7 = _mm_bf(sct_bf, s6)
    s8 = _mm_bf(sct_bf, s7)

    return _combine(g1, g2, g3, s1, s2, s4, s8, bcat)
